# baseline (speedup 1.0000x reference)
import jax
import jax.numpy as jnp
from jax.experimental import pallas as pl

NUM_Q = 4
K = 1024
D = 256
BETA = 0.25

BM = 768


def _rvq_body(z_ref, cb_ref, cbn_ref, zq_ref, idx_ref, loss_ref):
    r = z_ref[...]
    zq_acc = jnp.zeros_like(r)
    loss = jnp.float32(0.0)
    idx_cols = []
    for i in range(NUM_Q):
        cb = cb_ref[i]
        cbn = cbn_ref[...][i]
        rown = jnp.sum(r * r, axis=1, keepdims=True)
        mm = jax.lax.dot_general(
            r, cb, (((1,), (1,)), ((), ())),
            preferred_element_type=jnp.float32)
        dist = (rown + cbn[None, :]) - 2.0 * mm
        lane = jax.lax.broadcasted_iota(jnp.int32, (BM, K), 1)
        minval = jnp.min(dist, axis=1, keepdims=True)
        at_min = dist == minval
        idx = jnp.min(jnp.where(at_min, lane, K), axis=1).astype(jnp.int32)
        onehot = at_min & (lane == idx[:, None])
        zq_l = jax.lax.dot_general(
            onehot.astype(jnp.float32), cb, (((1,), (0,)), ((), ())),
            preferred_element_type=jnp.float32,
            precision=jax.lax.Precision.HIGHEST)
        r = r - zq_l
        zq_acc = zq_acc + zq_l
        diff = zq_l - r
        loss = loss + jnp.sum(diff * diff)
        idx_cols.append(idx)
    zq_ref[...] = zq_acc
    idx_ref[...] = jnp.stack(idx_cols, axis=1)

    @pl.when(pl.program_id(0) == 0)
    def _():
        loss_ref[...] = jnp.zeros((1, 1), jnp.float32)

    loss_ref[...] = loss_ref[...] + loss


def kernel(z, codebooks):
    B, L, _ = z.shape
    M = B * L
    zf = z.reshape(M, D)
    cbn = jnp.sum(codebooks ** 2, axis=2)
    grid = M // BM
    zq, idx, loss = pl.pallas_call(
        _rvq_body,
        grid=(grid,),
        in_specs=[
            pl.BlockSpec((BM, D), lambda i: (i, 0)),
            pl.BlockSpec((NUM_Q, K, D), lambda i: (0, 0, 0)),
            pl.BlockSpec((NUM_Q, K), lambda i: (0, 0)),
        ],
        out_specs=[
            pl.BlockSpec((BM, D), lambda i: (i, 0)),
            pl.BlockSpec((BM, NUM_Q), lambda i: (i, 0)),
            pl.BlockSpec((1, 1), lambda i: (0, 0)),
        ],
        out_shape=[
            jax.ShapeDtypeStruct((M, D), jnp.float32),
            jax.ShapeDtypeStruct((M, NUM_Q), jnp.int32),
            jax.ShapeDtypeStruct((1, 1), jnp.float32),
        ],
    )(zf, codebooks, cbn)
    zq = zq.reshape(B, L, D)
    z_q = z + (zq - z)
    all_indices = idx.reshape(B, L, NUM_Q).astype(jnp.int64)
    total_loss = (loss[0, 0] * ((1.0 + BETA) / (M * D))).astype(jnp.float32)
    return (z_q, all_indices, total_loss)


# speedup vs baseline: 1.5674x; 1.5674x over previous
import jax
import jax.numpy as jnp
from jax.experimental import pallas as pl

NUM_Q = 4
K = 1024
D = 256
BETA = 0.25

BM = 768


def _rvq_body(z_ref, cb_ref, cbn_ref, cbh_ref, cbm_ref, cbl_ref,
              zq_ref, idx_ref, loss_ref):
    r = z_ref[...]
    zq_acc = jnp.zeros_like(r)
    loss = jnp.float32(0.0)
    idx_cols = []
    for i in range(NUM_Q):
        cb = cb_ref[i]
        cbn = cbn_ref[...][i]
        rown = jnp.sum(r * r, axis=1, keepdims=True)
        mm = jax.lax.dot_general(
            r, cb, (((1,), (1,)), ((), ())),
            preferred_element_type=jnp.float32)
        dist = (rown + cbn[None, :]) - 2.0 * mm
        lane = jax.lax.broadcasted_iota(jnp.int32, (BM, K), 1)
        minval = jnp.min(dist, axis=1, keepdims=True)
        at_min = dist == minval
        idx = jnp.min(jnp.where(at_min, lane, K), axis=1).astype(jnp.int32)
        onehot = (at_min & (lane == idx[:, None])).astype(jnp.bfloat16)
        dn = (((1,), (0,)), ((), ()))
        g_h = jax.lax.dot_general(onehot, cbh_ref[i], dn,
                                  preferred_element_type=jnp.float32)
        g_m = jax.lax.dot_general(onehot, cbm_ref[i], dn,
                                  preferred_element_type=jnp.float32)
        g_l = jax.lax.dot_general(onehot, cbl_ref[i], dn,
                                  preferred_element_type=jnp.float32)
        zq_l = (g_h + g_m) + g_l
        r = r - zq_l
        zq_acc = zq_acc + zq_l
        diff = zq_l - r
        loss = loss + jnp.sum(diff * diff)
        idx_cols.append(idx)
    zq_ref[...] = zq_acc
    idx_ref[...] = jnp.stack(idx_cols, axis=1)

    @pl.when(pl.program_id(0) == 0)
    def _():
        loss_ref[...] = jnp.zeros((1, 1), jnp.float32)

    loss_ref[...] = loss_ref[...] + loss


def kernel(z, codebooks):
    B, L, _ = z.shape
    M = B * L
    zf = z.reshape(M, D)
    cbn = jnp.sum(codebooks ** 2, axis=2)
    cbh = codebooks.astype(jnp.bfloat16)
    r1 = codebooks - cbh.astype(jnp.float32)
    cbm = r1.astype(jnp.bfloat16)
    cbl = (r1 - cbm.astype(jnp.float32)).astype(jnp.bfloat16)
    grid = M // BM
    zq, idx, loss = pl.pallas_call(
        _rvq_body,
        grid=(grid,),
        in_specs=[
            pl.BlockSpec((BM, D), lambda i: (i, 0)),
            pl.BlockSpec((NUM_Q, K, D), lambda i: (0, 0, 0)),
            pl.BlockSpec((NUM_Q, K), lambda i: (0, 0)),
            pl.BlockSpec((NUM_Q, K, D), lambda i: (0, 0, 0)),
            pl.BlockSpec((NUM_Q, K, D), lambda i: (0, 0, 0)),
            pl.BlockSpec((NUM_Q, K, D), lambda i: (0, 0, 0)),
        ],
        out_specs=[
            pl.BlockSpec((BM, D), lambda i: (i, 0)),
            pl.BlockSpec((BM, NUM_Q), lambda i: (i, 0)),
            pl.BlockSpec((1, 1), lambda i: (0, 0)),
        ],
        out_shape=[
            jax.ShapeDtypeStruct((M, D), jnp.float32),
            jax.ShapeDtypeStruct((M, NUM_Q), jnp.int32),
            jax.ShapeDtypeStruct((1, 1), jnp.float32),
        ],
    )(zf, codebooks, cbn, cbh, cbm, cbl)
    zq = zq.reshape(B, L, D)
    z_q = z + (zq - z)
    all_indices = idx.reshape(B, L, NUM_Q).astype(jnp.int64)
    total_loss = (loss[0, 0] * ((1.0 + BETA) / (M * D))).astype(jnp.float32)
    return (z_q, all_indices, total_loss)


# speedup vs baseline: 2.4015x; 1.5322x over previous
import jax
import jax.numpy as jnp
from jax.experimental import pallas as pl

NUM_Q = 4
K = 1024
D = 256
BETA = 0.25

BM = 768


def _rvq_body(z_ref, cb_ref, cbn_ref, cbh_ref,
              zq_ref, idx_ref, loss_ref):
    r = z_ref[...]
    zq_acc = jnp.zeros_like(r)
    loss = jnp.float32(0.0)
    idx_cols = []
    for i in range(NUM_Q):
        cb = cb_ref[i]
        cbn = cbn_ref[...][i]
        rown = jnp.sum(r * r, axis=1, keepdims=True)
        mm = jax.lax.dot_general(
            r, cb, (((1,), (1,)), ((), ())),
            preferred_element_type=jnp.float32)
        dist = (rown + cbn[None, :]) - 2.0 * mm
        lane = jax.lax.broadcasted_iota(jnp.int32, (BM, K), 1)
        minval = jnp.min(dist, axis=1, keepdims=True)
        at_min = dist == minval
        idx = jnp.min(jnp.where(at_min, lane, K), axis=1).astype(jnp.int32)
        onehot = (lane == idx[:, None]).astype(jnp.bfloat16)
        dn = (((1,), (0,)), ((), ()))
        zq_l = jax.lax.dot_general(onehot, cbh_ref[i], dn,
                                   preferred_element_type=jnp.float32)
        r = r - zq_l
        zq_acc = zq_acc + zq_l
        diff = zq_l - r
        loss = loss + jnp.sum(diff * diff)
        idx_cols.append(idx)
    zq_ref[...] = zq_acc
    idx_ref[...] = jnp.stack(idx_cols, axis=1)

    @pl.when(pl.program_id(0) == 0)
    def _():
        loss_ref[...] = jnp.zeros((1, 1), jnp.float32)

    loss_ref[...] = loss_ref[...] + loss


def kernel(z, codebooks):
    B, L, _ = z.shape
    M = B * L
    zf = z.reshape(M, D)
    cbn = jnp.sum(codebooks ** 2, axis=2)
    cbh = codebooks.astype(jnp.bfloat16)
    grid = M // BM
    zq, idx, loss = pl.pallas_call(
        _rvq_body,
        grid=(grid,),
        in_specs=[
            pl.BlockSpec((BM, D), lambda i: (i, 0)),
            pl.BlockSpec((NUM_Q, K, D), lambda i: (0, 0, 0)),
            pl.BlockSpec((NUM_Q, K), lambda i: (0, 0)),
            pl.BlockSpec((NUM_Q, K, D), lambda i: (0, 0, 0)),
        ],
        out_specs=[
            pl.BlockSpec((BM, D), lambda i: (i, 0)),
            pl.BlockSpec((BM, NUM_Q), lambda i: (i, 0)),
            pl.BlockSpec((1, 1), lambda i: (0, 0)),
        ],
        out_shape=[
            jax.ShapeDtypeStruct((M, D), jnp.float32),
            jax.ShapeDtypeStruct((M, NUM_Q), jnp.int32),
            jax.ShapeDtypeStruct((1, 1), jnp.float32),
        ],
    )(zf, codebooks, cbn, cbh)
    zq = zq.reshape(B, L, D)
    z_q = z + (zq - z)
    all_indices = idx.reshape(B, L, NUM_Q).astype(jnp.int64)
    total_loss = (loss[0, 0] * ((1.0 + BETA) / (M * D))).astype(jnp.float32)
    return (z_q, all_indices, total_loss)


# speedup vs baseline: 2.5333x; 1.0549x over previous
import jax
import jax.numpy as jnp
from jax.experimental import pallas as pl

NUM_Q = 4
K = 1024
D = 256
BETA = 0.25

BM = 1152


def _rvq_body(z_ref, cb_ref, cbn_ref, cbh_ref,
              zq_ref, idx_ref, loss_ref):
    r = z_ref[...]
    zq_acc = jnp.zeros_like(r)
    loss = jnp.float32(0.0)
    idx_cols = []
    for i in range(NUM_Q):
        cb = cb_ref[i]
        cbn = cbn_ref[...][i]
        rown = jnp.sum(r * r, axis=1, keepdims=True)
        mm2 = jax.lax.dot_general(
            r + r, cb, (((1,), (1,)), ((), ())),
            preferred_element_type=jnp.float32)
        dist = (rown + cbn[None, :]) - mm2
        lane = jax.lax.broadcasted_iota(jnp.int32, (BM, K), 1)
        minval = jnp.min(dist, axis=1, keepdims=True)
        at_min = dist == minval
        idx = jnp.min(jnp.where(at_min, lane, K), axis=1)
        onehot = (lane == idx[:, None]).astype(jnp.bfloat16)
        dn = (((1,), (0,)), ((), ()))
        zq_l = jax.lax.dot_general(onehot, cbh_ref[i], dn,
                                   preferred_element_type=jnp.float32)
        r = r - zq_l
        zq_acc = zq_acc + zq_l
        diff = zq_l - r
        loss = loss + jnp.sum(diff * diff)
        idx_cols.append(idx)
    zq_ref[...] = zq_acc
    idx_ref[...] = jnp.stack(idx_cols, axis=1)

    @pl.when(pl.program_id(0) == 0)
    def _():
        loss_ref[...] = jnp.zeros((1, 1), jnp.float32)

    loss_ref[...] = loss_ref[...] + loss


def kernel(z, codebooks):
    B, L, _ = z.shape
    M = B * L
    zf = z.reshape(M, D)
    cbn = jnp.sum(codebooks ** 2, axis=2)
    cbh = codebooks.astype(jnp.bfloat16)
    grid = M // BM
    zq, idx, loss = pl.pallas_call(
        _rvq_body,
        grid=(grid,),
        in_specs=[
            pl.BlockSpec((BM, D), lambda i: (i, 0)),
            pl.BlockSpec((NUM_Q, K, D), lambda i: (0, 0, 0)),
            pl.BlockSpec((NUM_Q, K), lambda i: (0, 0)),
            pl.BlockSpec((NUM_Q, K, D), lambda i: (0, 0, 0)),
        ],
        out_specs=[
            pl.BlockSpec((BM, D), lambda i: (i, 0)),
            pl.BlockSpec((BM, NUM_Q), lambda i: (i, 0)),
            pl.BlockSpec((1, 1), lambda i: (0, 0)),
        ],
        out_shape=[
            jax.ShapeDtypeStruct((M, D), jnp.float32),
            jax.ShapeDtypeStruct((M, NUM_Q), jnp.int32),
            jax.ShapeDtypeStruct((1, 1), jnp.float32),
        ],
    )(zf, codebooks, cbn, cbh)
    zq = zq.reshape(B, L, D)
    z_q = z + (zq - z)
    all_indices = idx.reshape(B, L, NUM_Q).astype(jnp.int64)
    total_loss = (loss[0, 0] * ((1.0 + BETA) / (M * D))).astype(jnp.float32)
    return (z_q, all_indices, total_loss)
